# baseline (speedup 1.0000x reference)
import functools

import jax
import jax.numpy as jnp
from jax import lax
from jax.experimental import pallas as pl
from jax.experimental.pallas import tpu as pltpu
from jax.experimental.pallas import tpu_sc as plsc

_N_TABLES = 8
_VOCAB = 100000
_DIM = 64
_BATCH = 4096
_TOTAL = _N_TABLES * _BATCH

_info = plsc.get_sparse_core_info()
_NC, _NS, _L = _info.num_cores, _info.num_subcores, _info.num_lanes
_NW = _NC * _NS
_W_PER_TABLE = _NW // _N_TABLES
_D_PER_W = _DIM // _W_PER_TABLE

_SPLIT = 49920
_ALIGNED = 99840
_TAIL = _VOCAB - _ALIGNED
_NCHUNK = _BATCH // _L


@functools.partial(
    pl.kernel,
    out_type=jax.ShapeDtypeStruct((_DIM, _TOTAL), jnp.float32),
    mesh=plsc.VectorSubcoreMesh(core_axis_name="c", subcore_axis_name="s"),
    scratch_types=[
        pltpu.VMEM((_BATCH,), jnp.int32),
        pltpu.VMEM((_VOCAB,), jnp.float32),
        pltpu.VMEM((_TAIL,), jnp.float32),
        pltpu.VMEM((_BATCH,), jnp.float32),
        pltpu.VMEM((_BATCH,), jnp.float32),
        pltpu.SemaphoreType.DMA,
        pltpu.SemaphoreType.DMA,
        pltpu.SemaphoreType.DMA,
        pltpu.SemaphoreType.DMA,
        pltpu.SemaphoreType.DMA,
    ],
    compiler_params=pltpu.CompilerParams(
        use_tc_tiling_on_sc=True, needs_layout_passes=False
    ),
)
def _gather_kernel(
    ids_hbm, tables_hbm, out_hbm,
    ids_v, col_v, tail_v, out0_v, out1_v,
    sem_lo, sem_hi, sem_tl, sem_o0, sem_o1,
):
    wid = lax.axis_index("s") * _NC + lax.axis_index("c")
    t = wid // _W_PER_TABLE
    d0 = (wid % _W_PER_TABLE) * _D_PER_W

    def _stage(k):
        row = tables_hbm.at[t, d0 + k]
        return (
            pltpu.async_copy(
                row.at[pl.ds(0, _SPLIT)], col_v.at[pl.ds(0, _SPLIT)], sem_lo
            ),
            pltpu.async_copy(
                row.at[pl.ds(_SPLIT, _SPLIT)],
                col_v.at[pl.ds(_SPLIT, _SPLIT)],
                sem_hi,
            ),
            pltpu.async_copy(row.at[pl.ds(_ALIGNED, _TAIL)], tail_v, sem_tl),
        )

    def _splice_tail():
        for i in range(_TAIL // _L):
            col_v[pl.ds(_ALIGNED + i * _L, _L)] = tail_v[pl.ds(i * _L, _L)]

    cps = _stage(0)
    pltpu.sync_copy(ids_hbm.at[t], ids_v)

    def _gather(out_v):
        def _body(i):
            sl = pl.ds(i * _L, _L)
            out_v[sl] = plsc.load_gather(col_v, [ids_v[sl]])

        plsc.parallel_loop(0, _NCHUNK, 1, unroll=4)(_body)

    out_bufs = (out0_v, out1_v)
    out_sems = (sem_o0, sem_o1)
    out_copies = [None, None]

    for k in range(_D_PER_W):
        out_v = out_bufs[k % 2]
        if out_copies[k % 2] is not None:
            out_copies[k % 2].wait()
        for cp in cps:
            cp.wait()
        _splice_tail()
        _gather(out_v)
        if k + 1 < _D_PER_W:
            cps = _stage(k + 1)
        out_copies[k % 2] = pltpu.async_copy(
            out_v, out_hbm.at[d0 + k, pl.ds(t * _BATCH, _BATCH)], out_sems[k % 2]
        )
    for c in out_copies:
        c.wait()


def kernel(ids, tables):
    tables_t = tables.transpose(0, 2, 1)
    out_t = _gather_kernel(ids, tables_t)
    return out_t.T


# speedup vs baseline: 1.0297x; 1.0297x over previous
import functools

import jax
import jax.numpy as jnp
from jax import lax
from jax.experimental import pallas as pl
from jax.experimental.pallas import tpu as pltpu
from jax.experimental.pallas import tpu_sc as plsc

_N_TABLES = 8
_VOCAB = 100000
_DIM = 64
_BATCH = 4096
_TOTAL = _N_TABLES * _BATCH

_info = plsc.get_sparse_core_info()
_NC, _NS, _L = _info.num_cores, _info.num_subcores, _info.num_lanes
_NW = _NC * _NS
_W_PER_TABLE = _NW // _N_TABLES
_D_PER_W = _DIM // _W_PER_TABLE

_NCHUNK = _BATCH // _L


@functools.partial(
    pl.kernel,
    out_type=jax.ShapeDtypeStruct((_DIM, _TOTAL), jnp.float32),
    mesh=plsc.VectorSubcoreMesh(core_axis_name="c", subcore_axis_name="s"),
    scratch_types=[
        pltpu.VMEM((_BATCH,), jnp.int32),
        pltpu.VMEM((_VOCAB,), jnp.float32),
        pltpu.VMEM((_BATCH,), jnp.float32),
        pltpu.VMEM((_BATCH,), jnp.float32),
        pltpu.SemaphoreType.DMA,
        pltpu.SemaphoreType.DMA,
        pltpu.SemaphoreType.DMA,
    ],
    compiler_params=pltpu.CompilerParams(
        use_tc_tiling_on_sc=True, needs_layout_passes=False
    ),
)
def _gather_kernel(
    ids_hbm, tables_hbm, out_hbm,
    ids_v, col_v, out0_v, out1_v,
    sem_st, sem_o0, sem_o1,
):
    wid = lax.axis_index("s") * _NC + lax.axis_index("c")
    t = wid // _W_PER_TABLE
    d0 = (wid % _W_PER_TABLE) * _D_PER_W

    def _stage(k):
        return (pltpu.async_copy(tables_hbm.at[t, d0 + k], col_v, sem_st),)

    cps = _stage(0)
    pltpu.sync_copy(ids_hbm.at[t], ids_v)

    def _gather(out_v):
        def _body(i):
            sl = pl.ds(i * _L, _L)
            out_v[sl] = plsc.load_gather(col_v, [ids_v[sl]])

        plsc.parallel_loop(0, _NCHUNK, 1, unroll=4)(_body)

    out_bufs = (out0_v, out1_v)
    out_sems = (sem_o0, sem_o1)
    out_copies = [None, None]

    for k in range(_D_PER_W):
        out_v = out_bufs[k % 2]
        if out_copies[k % 2] is not None:
            out_copies[k % 2].wait()
        for cp in cps:
            cp.wait()
        _gather(out_v)
        if k + 1 < _D_PER_W:
            cps = _stage(k + 1)
        out_copies[k % 2] = pltpu.async_copy(
            out_v, out_hbm.at[d0 + k, pl.ds(t * _BATCH, _BATCH)], out_sems[k % 2]
        )
    for c in out_copies:
        c.wait()


def kernel(ids, tables):
    tables_t = tables.transpose(0, 2, 1)
    out_t = _gather_kernel(ids, tables_t)
    return out_t.T
